# Initial kernel scaffold; baseline (speedup 1.0000x reference)
import jax
import jax.numpy as jnp
from jax.experimental import pallas as pl


def kernel(x, adj, W, a_src, a_dst):
    raise NotImplementedError("write your pallas kernel here")


# speedup vs baseline: 1.5299x; 1.5299x over previous
import functools

import jax
import jax.numpy as jnp
from jax import lax
from jax.experimental import pallas as pl

_SLOPE = 0.2
_N_HEADS = 4
_NEG = -9e15


def _proj_kernel(x_ref, w_ref, h_ref):
    h_ref[0] = jnp.dot(x_ref[...], w_ref[0], preferred_element_type=jnp.float32)


def _attn_kernel(h_all_ref, h_blk_ref, adj_ref, x_ref, asrc_ref, adst_ref, y_ref):
    x_blk = x_ref[...]
    adj_ok = adj_ref[...].astype(jnp.int32) > 0
    acc = jnp.zeros_like(x_blk)
    for hi in range(_N_HEADS):
        h_full = h_all_ref[hi]
        h_blk = h_blk_ref[hi]
        f1 = lax.dot_general(h_blk, asrc_ref[hi][None, :],
                             (((1,), (1,)), ((), ())),
                             preferred_element_type=jnp.float32)
        f2 = lax.dot_general(adst_ref[hi][None, :], h_full,
                             (((1,), (1,)), ((), ())),
                             preferred_element_type=jnp.float32)
        e = f1 + f2
        e = jnp.where(e >= 0, e, _SLOPE * e)
        e = jnp.where(adj_ok, e, _NEG)
        m = jnp.max(e, axis=-1, keepdims=True)
        p = jnp.exp(e - m)
        s = jnp.sum(p, axis=-1, keepdims=True)
        out = jnp.dot(p, h_full, preferred_element_type=jnp.float32) / s
        v = out + x_blk
        acc = acc + jnp.where(v > 0, v, jnp.exp(jnp.minimum(v, 0.0)) - 1.0)
    y_ref[...] = acc * (1.0 / _N_HEADS)


def _gram_kernel(y_all_ref, y_blk_ref, out_ref):
    out_ref[...] = lax.dot_general(y_all_ref[...], y_blk_ref[...],
                                   (((1,), (1,)), ((), ())),
                                   preferred_element_type=jnp.float32)


@functools.partial(jax.jit, static_argnames=())
def kernel(x, adj, W, a_src, a_dst):
    N, D = x.shape
    H = W.shape[0]
    B = 256
    GB = 512

    adj_i8 = adj.astype(jnp.int8)

    proj = pl.pallas_call(
        _proj_kernel,
        grid=(H,),
        in_specs=[
            pl.BlockSpec((N, D), lambda i: (0, 0)),
            pl.BlockSpec((1, D, D), lambda i: (i, 0, 0)),
        ],
        out_specs=pl.BlockSpec((1, N, D), lambda i: (i, 0, 0)),
        out_shape=jax.ShapeDtypeStruct((H, N, D), jnp.float32),
    )

    attn = pl.pallas_call(
        _attn_kernel,
        grid=(N // B,),
        in_specs=[
            pl.BlockSpec((H, N, D), lambda i: (0, 0, 0)),
            pl.BlockSpec((H, B, D), lambda i: (0, i, 0)),
            pl.BlockSpec((B, N), lambda i: (i, 0)),
            pl.BlockSpec((B, D), lambda i: (i, 0)),
            pl.BlockSpec((H, D), lambda i: (0, 0)),
            pl.BlockSpec((H, D), lambda i: (0, 0)),
        ],
        out_specs=pl.BlockSpec((B, D), lambda i: (i, 0)),
        out_shape=jax.ShapeDtypeStruct((N, D), jnp.float32),
    )

    gram = pl.pallas_call(
        _gram_kernel,
        grid=(N // GB,),
        in_specs=[
            pl.BlockSpec((N, D), lambda i: (0, 0)),
            pl.BlockSpec((GB, D), lambda i: (i, 0)),
        ],
        out_specs=pl.BlockSpec((N, GB), lambda i: (0, i)),
        out_shape=jax.ShapeDtypeStruct((N, N), jnp.float32),
    )

    y = x
    for _ in range(3):
        h = proj(y, W)
        y = attn(h, h, adj_i8, y, a_src, a_dst)
    return gram(y, y)


# speedup vs baseline: 2.2909x; 1.4974x over previous
import functools

import jax
import jax.numpy as jnp
from jax import lax
from jax.experimental import pallas as pl

_SLOPE = 0.2
_N_HEADS = 4
_LOG2E = 1.4426950408889634


def _proj_kernel(x_ref, w_ref, h_ref):
    h_ref[0] = jnp.dot(x_ref[...], w_ref[0], preferred_element_type=jnp.float32)


def _attn_kernel(h_all_ref, h_blk_ref, adj_ref, x_ref, asrc_ref, adst_ref, y_ref):
    x_blk = x_ref[...]
    adj_blk = adj_ref[...]
    acc = jnp.zeros_like(x_blk)
    for hi in range(_N_HEADS):
        h_full = h_all_ref[hi]
        h_blk = h_blk_ref[hi]
        f1 = lax.dot_general(h_blk, asrc_ref[hi][None, :] * _LOG2E,
                             (((1,), (1,)), ((), ())),
                             preferred_element_type=jnp.float32)
        f2 = lax.dot_general(adst_ref[hi][None, :] * _LOG2E, h_full,
                             (((1,), (1,)), ((), ())),
                             preferred_element_type=jnp.float32)
        e = f1 + f2
        e = jnp.maximum(e, _SLOPE * e)
        p = jnp.exp2(e) * adj_blk
        s = jnp.sum(p, axis=-1, keepdims=True)
        out = jnp.dot(p, h_full, preferred_element_type=jnp.float32) / s
        v = out + x_blk
        acc = acc + jnp.where(v > 0, v, jnp.exp(jnp.minimum(v, 0.0)) - 1.0)
    y_ref[...] = acc * (1.0 / _N_HEADS)


def _gram_kernel(y_all_ref, y_blk_ref, out_ref):
    out_ref[...] = lax.dot_general(y_all_ref[...], y_blk_ref[...],
                                   (((1,), (1,)), ((), ())),
                                   preferred_element_type=jnp.float32)


@functools.partial(jax.jit, static_argnames=())
def kernel(x, adj, W, a_src, a_dst):
    N, D = x.shape
    H = W.shape[0]
    B = 256
    GB = 512

    proj = pl.pallas_call(
        _proj_kernel,
        grid=(H,),
        in_specs=[
            pl.BlockSpec((N, D), lambda i: (0, 0)),
            pl.BlockSpec((1, D, D), lambda i: (i, 0, 0)),
        ],
        out_specs=pl.BlockSpec((1, N, D), lambda i: (i, 0, 0)),
        out_shape=jax.ShapeDtypeStruct((H, N, D), jnp.float32),
    )

    attn = pl.pallas_call(
        _attn_kernel,
        grid=(N // B,),
        in_specs=[
            pl.BlockSpec((H, N, D), lambda i: (0, 0, 0)),
            pl.BlockSpec((H, B, D), lambda i: (0, i, 0)),
            pl.BlockSpec((B, N), lambda i: (i, 0)),
            pl.BlockSpec((B, D), lambda i: (i, 0)),
            pl.BlockSpec((H, D), lambda i: (0, 0)),
            pl.BlockSpec((H, D), lambda i: (0, 0)),
        ],
        out_specs=pl.BlockSpec((B, D), lambda i: (i, 0)),
        out_shape=jax.ShapeDtypeStruct((N, D), jnp.float32),
    )

    gram = pl.pallas_call(
        _gram_kernel,
        grid=(N // GB,),
        in_specs=[
            pl.BlockSpec((N, D), lambda i: (0, 0)),
            pl.BlockSpec((GB, D), lambda i: (i, 0)),
        ],
        out_specs=pl.BlockSpec((N, GB), lambda i: (0, i)),
        out_shape=jax.ShapeDtypeStruct((N, N), jnp.float32),
    )

    y = x
    for _ in range(3):
        h = proj(y, W)
        y = attn(h, h, adj, y, a_src, a_dst)
    return gram(y, y)
